# baseline (speedup 1.0000x reference)
import jax
import jax.numpy as jnp
from jax import lax
from jax.experimental import pallas as pl
from jax.experimental.pallas import tpu as pltpu
from jax.experimental.pallas import tpu_sc as plsc

_NC = 2
_NS = 16
_NW = _NC * _NS
_B = 16384
_D = 32
_GROUP = 128 // _D
_BPW = _B // _NW
_CHUNK = 128
_NCHUNK = _BPW // _CHUNK


def _sc_entry(table_hbm, qi_hbm, qj_hbm, oi_hbm, oj_hbm, beta_hbm, out_hbm,
              q_i, q_j, off_v, off_i, off_j, wide_i, wide_j,
              sums_v, out_v, beta_v, sem0, sem1):
    cid = lax.axis_index("c")
    sid = lax.axis_index("s")
    wid = sid * _NC + cid
    base = wid * _BPW

    pltpu.sync_copy(qi_hbm.at[wid], q_i)
    pltpu.sync_copy(qj_hbm.at[wid], q_j)
    pltpu.sync_copy(oi_hbm.at[wid], off_i)
    pltpu.sync_copy(oj_hbm.at[wid], off_j)
    pltpu.sync_copy(beta_hbm, beta_v)

    sems = (sem0, sem1)

    def fire(k):
        s = sems[k % 2]
        return (pltpu.async_copy(table_hbm.at[q_i.at[k]], wide_i.at[k % 2], s),
                pltpu.async_copy(table_hbm.at[q_j.at[k]], wide_j.at[k % 2], s))

    def quarter(k, copies):
        for cp in copies:
            cp.wait()
        bi = wide_i.at[k % 2]
        bj = wide_j.at[k % 2]

        def group_body(i, carry):
            r16 = i * 16
            offv_i = off_i[k, pl.ds(r16, 16)]
            offv_j = off_j[k, pl.ds(r16, 16)]
            base_line = k * (_CHUNK // 8) + i * 2
            for l in range(16):
                r = r16 + l
                a = offv_i[l]
                b = offv_j[l]
                zi0 = bi[r, pl.ds(a, 16)]
                zi1 = bi[r, pl.ds(a + 16, 16)]
                zj0 = bj[r, pl.ds(b, 16)]
                zj1 = bj[r, pl.ds(b + 16, 16)]
                d0 = zi0 - zj0
                d1 = zi1 - zj1
                s2 = d0 * d0 + d1 * d1
                sums_v[base_line + (l >> 3),
                       pl.ds((l & 7) * 16, 16)] = plsc.cumsum(s2)
            return carry

        lax.fori_loop(0, _CHUNK // 16, group_body, 0)

    c0 = fire(0)
    c1 = fire(1)
    quarter(0, c0)
    c2 = fire(2)
    quarter(1, c1)
    c3 = fire(3)
    quarter(2, c2)
    quarter(3, c3)

    beta_vec = beta_v[...]
    lane = jax.lax.broadcasted_iota(jnp.int32, (16,), 0)
    last = jnp.full((16,), 15, jnp.int32)
    for g in range(_BPW // 16):
        rows = g * 16 + lane
        x = jnp.maximum(
            plsc.load_gather(
                sums_v,
                [lax.shift_right_logical(rows, 3), (rows & 7) * 16 + last]),
            1e-12)
        i = plsc.bitcast(x, jnp.int32)
        i = 0x5F3759DF - lax.shift_right_arithmetic(i, 1)
        r = plsc.bitcast(i, jnp.float32)
        half = 0.5 * x
        for _ in range(3):
            r = r * (1.5 - half * r * r)
        out_v[pl.ds(g * 16, 16)] = beta_vec - x * r

    pltpu.sync_copy(out_v, out_hbm.at[pl.ds(base, _BPW)])


def kernel(node_i, node_j, table, beta):
    mesh = plsc.VectorSubcoreMesh(core_axis_name="c", subcore_axis_name="s")
    k = pl.kernel(
        _sc_entry,
        out_type=jax.ShapeDtypeStruct((_B,), jnp.float32),
        mesh=mesh,
        compiler_params=pltpu.CompilerParams(needs_layout_passes=False),
        scratch_types=[
            pltpu.VMEM((_NCHUNK, _CHUNK), jnp.int32),
            pltpu.VMEM((_NCHUNK, _CHUNK), jnp.int32),
            pltpu.VMEM((_NCHUNK, _CHUNK), jnp.int32),
            pltpu.VMEM((_NCHUNK, _CHUNK), jnp.int32),
            pltpu.VMEM((_NCHUNK, _CHUNK), jnp.int32),
            pltpu.VMEM((2, _CHUNK, 128), jnp.float32),
            pltpu.VMEM((2, _CHUNK, 128), jnp.float32),
            pltpu.VMEM((_BPW // 8, 128), jnp.float32),
            pltpu.VMEM((_BPW,), jnp.float32),
            pltpu.VMEM((16,), jnp.float32),
            pltpu.SemaphoreType.DMA,
            pltpu.SemaphoreType.DMA,
        ],
    )
    t4 = table.reshape(1000000 // _GROUP, _D * _GROUP)
    qi = (node_i >> 2).reshape(_NW, _NCHUNK, _CHUNK)
    qj = (node_j >> 2).reshape(_NW, _NCHUNK, _CHUNK)
    oi = ((node_i & 3) * _D).reshape(_NW, _NCHUNK, _CHUNK)
    oj = ((node_j & 3) * _D).reshape(_NW, _NCHUNK, _CHUNK)
    beta_vec = jnp.broadcast_to(beta.astype(jnp.float32), (16,))
    return k(t4, qi, qj, oi, oj, beta_vec)


# speedup vs baseline: 1.0008x; 1.0008x over previous
import jax
import jax.numpy as jnp
from jax import lax
from jax.experimental import pallas as pl
from jax.experimental.pallas import tpu as pltpu
from jax.experimental.pallas import tpu_sc as plsc

_NC = 2
_NS = 16
_NW = _NC * _NS
_B = 16384
_D = 32
_GROUP = 128 // _D
_BPW = _B // _NW
_CHUNK = 128
_NCHUNK = _BPW // _CHUNK


def _sc_entry(table_hbm, qi_hbm, qj_hbm, oi_hbm, oj_hbm, beta_hbm, out_hbm,
              q_i, q_j, off_v, off_i, off_j, wide_i, wide_j,
              sums_v, out_v, beta_v, sem0, sem1):
    cid = lax.axis_index("c")
    sid = lax.axis_index("s")
    wid = sid * _NC + cid
    base = wid * _BPW

    pltpu.sync_copy(qi_hbm.at[wid], q_i)
    pltpu.sync_copy(qj_hbm.at[wid], q_j)
    pltpu.sync_copy(oi_hbm.at[wid], off_i)
    pltpu.sync_copy(oj_hbm.at[wid], off_j)
    pltpu.sync_copy(beta_hbm, beta_v)

    sems = (sem0, sem1)

    def fire(k):
        s = sems[k % 2]
        return (pltpu.async_copy(table_hbm.at[q_i.at[k]], wide_i.at[k % 2], s),
                pltpu.async_copy(table_hbm.at[q_j.at[k]], wide_j.at[k % 2], s))

    def quarter(k, copies):
        for cp in copies:
            cp.wait()
        bi = wide_i.at[k % 2]
        bj = wide_j.at[k % 2]

        def group_body(i, carry):
            r16 = i * 16
            offv_i = off_i[k, pl.ds(r16, 16)]
            offv_j = off_j[k, pl.ds(r16, 16)]
            base_line = k * (_CHUNK // 8) + i * 2
            for l in range(16):
                r = r16 + l
                a = offv_i[l]
                b = offv_j[l]
                zi0 = bi[r, pl.ds(a, 16)]
                zi1 = bi[r, pl.ds(a + 16, 16)]
                zj0 = bj[r, pl.ds(b, 16)]
                zj1 = bj[r, pl.ds(b + 16, 16)]
                d0 = zi0 - zj0
                d1 = zi1 - zj1
                s2 = d0 * d0 + d1 * d1
                sums_v[base_line + (l >> 3),
                       pl.ds((l & 7) * 16, 16)] = plsc.cumsum(s2)
            return carry

        lax.fori_loop(0, _CHUNK // 16, group_body, 0)

    c0 = fire(0)
    c1 = fire(1)
    quarter(0, c0)
    c2 = fire(2)
    quarter(1, c1)
    c3 = fire(3)
    quarter(2, c2)
    quarter(3, c3)

    beta_vec = beta_v[...]
    lane = jax.lax.broadcasted_iota(jnp.int32, (16,), 0)
    last = jnp.full((16,), 15, jnp.int32)
    for g in range(_BPW // 16):
        rows = g * 16 + lane
        x = jnp.maximum(
            plsc.load_gather(
                sums_v,
                [lax.shift_right_logical(rows, 3), (rows & 7) * 16 + last]),
            1e-12)
        i = plsc.bitcast(x, jnp.int32)
        i = 0x5F3759DF - lax.shift_right_arithmetic(i, 1)
        r = plsc.bitcast(i, jnp.float32)
        half = 0.5 * x
        for _ in range(3):
            r = r * (1.5 - half * r * r)
        out_v[pl.ds(g * 16, 16)] = beta_vec - x * r

    pltpu.sync_copy(out_v, out_hbm.at[pl.ds(base, _BPW)])


def kernel(node_i, node_j, table, beta):
    mesh = plsc.VectorSubcoreMesh(core_axis_name="c", subcore_axis_name="s")
    k = pl.kernel(
        _sc_entry,
        out_type=jax.ShapeDtypeStruct((_B,), jnp.float32),
        mesh=mesh,
        compiler_params=pltpu.CompilerParams(
            needs_layout_passes=False, skip_device_barrier=True),
        scratch_types=[
            pltpu.VMEM((_NCHUNK, _CHUNK), jnp.int32),
            pltpu.VMEM((_NCHUNK, _CHUNK), jnp.int32),
            pltpu.VMEM((_NCHUNK, _CHUNK), jnp.int32),
            pltpu.VMEM((_NCHUNK, _CHUNK), jnp.int32),
            pltpu.VMEM((_NCHUNK, _CHUNK), jnp.int32),
            pltpu.VMEM((2, _CHUNK, 128), jnp.float32),
            pltpu.VMEM((2, _CHUNK, 128), jnp.float32),
            pltpu.VMEM((_BPW // 8, 128), jnp.float32),
            pltpu.VMEM((_BPW,), jnp.float32),
            pltpu.VMEM((16,), jnp.float32),
            pltpu.SemaphoreType.DMA,
            pltpu.SemaphoreType.DMA,
        ],
    )
    t4 = table.reshape(1000000 // _GROUP, _D * _GROUP)
    qi = (node_i >> 2).reshape(_NW, _NCHUNK, _CHUNK)
    qj = (node_j >> 2).reshape(_NW, _NCHUNK, _CHUNK)
    oi = ((node_i & 3) * _D).reshape(_NW, _NCHUNK, _CHUNK)
    oj = ((node_j & 3) * _D).reshape(_NW, _NCHUNK, _CHUNK)
    beta_vec = jnp.broadcast_to(beta.astype(jnp.float32), (16,))
    return k(t4, qi, qj, oi, oj, beta_vec)


# speedup vs baseline: 2.1079x; 2.1062x over previous
import jax
import jax.numpy as jnp
from jax import lax
from jax.experimental import pallas as pl
from jax.experimental.pallas import tpu as pltpu
from jax.experimental.pallas import tpu_sc as plsc

_NC = 2
_NS = 16
_NW = _NC * _NS
_B = 16384
_D = 32
_BPW = _B // _NW
_CHUNK = 128
_NCHUNK = _BPW // _CHUNK
_PPR = 4
_NROUND = _BPW // _PPR


def _sc_entry(tab_hbm, ni_hbm, nj_hbm, beta_hbm, out_hbm,
              idx_i, idx_j, blk, sums_v, out_v, beta_v, sem0, sem1):
    cid = lax.axis_index("c")
    sid = lax.axis_index("s")
    wid = sid * _NC + cid
    base = wid * _BPW

    pltpu.sync_copy(ni_hbm.at[wid], idx_i)
    pltpu.sync_copy(nj_hbm.at[wid], idx_j)
    pltpu.sync_copy(beta_hbm, beta_v)

    sems = (sem0, sem1)
    lane = jax.lax.broadcasted_iota(jnp.int32, (16,), 0)

    def loadidx(g):
        c = lax.div(g, _CHUNK // _PPR)
        lo = lax.rem(g, _CHUNK // _PPR) * _PPR
        iv = idx_i[c, pl.ds(lo, 16)]
        jv = idx_j[c, pl.ds(lo, 16)]
        return iv, jv

    def fire(g, k):
        iv, jv = loadidx(g)
        tc_i = lax.shift_right_logical(iv, 7) * 128
        tc_j = lax.shift_right_logical(jv, 7) * 128
        for p in range(_PPR):
            pltpu.async_copy(
                tab_hbm.at[:, pl.ds(pl.multiple_of(tc_i[p], 128), 128)],
                blk.at[k, 2 * p], sems[k])
            pltpu.async_copy(
                tab_hbm.at[:, pl.ds(pl.multiple_of(tc_j[p], 128), 128)],
                blk.at[k, 2 * p + 1], sems[k])

    def consume(g, k):
        for p in range(2 * _PPR):
            pltpu.make_async_copy(
                tab_hbm.at[:, pl.ds(0, 128)], blk.at[k, p], sems[k]).wait()
        iv, jv = loadidx(g)
        col_i = iv & 127
        col_j = jv & 127
        for p in range(_PPR):
            ci = jnp.broadcast_to(col_i[p], (16,))
            cj = jnp.broadcast_to(col_j[p], (16,))
            zi0 = plsc.load_gather(blk.at[k, 2 * p], [lane, ci])
            zi1 = plsc.load_gather(blk.at[k, 2 * p], [lane + 16, ci])
            zj0 = plsc.load_gather(blk.at[k, 2 * p + 1], [lane, cj])
            zj1 = plsc.load_gather(blk.at[k, 2 * p + 1], [lane + 16, cj])
            d0 = zi0 - zj0
            d1 = zi1 - zj1
            s2 = d0 * d0 + d1 * d1
            rr = g * _PPR + p
            sums_v[lax.shift_right_logical(rr, 3),
                   pl.ds((rr & 7) * 16, 16)] = plsc.cumsum(s2)

    fire(0, 0)

    def round_pair(q, carry):
        g = q * 2
        fire(g + 1, 1)
        consume(g, 0)

        @pl.when(q < _NROUND // 2 - 1)
        def _():
            fire(g + 2, 0)

        consume(g + 1, 1)
        return carry

    lax.fori_loop(0, _NROUND // 2, round_pair, 0)

    beta_vec = beta_v[...]
    last = jnp.full((16,), 15, jnp.int32)
    for g in range(_BPW // 16):
        rows = g * 16 + lane
        x = jnp.maximum(
            plsc.load_gather(
                sums_v,
                [lax.shift_right_logical(rows, 3), (rows & 7) * 16 + last]),
            1e-12)
        i = plsc.bitcast(x, jnp.int32)
        i = 0x5F3759DF - lax.shift_right_arithmetic(i, 1)
        r = plsc.bitcast(i, jnp.float32)
        half = 0.5 * x
        for _ in range(3):
            r = r * (1.5 - half * r * r)
        out_v[pl.ds(g * 16, 16)] = beta_vec - x * r

    pltpu.sync_copy(out_v, out_hbm.at[pl.ds(base, _BPW)])


def kernel(node_i, node_j, table, beta):
    mesh = plsc.VectorSubcoreMesh(core_axis_name="c", subcore_axis_name="s")
    k = pl.kernel(
        _sc_entry,
        out_type=jax.ShapeDtypeStruct((_B,), jnp.float32),
        mesh=mesh,
        compiler_params=pltpu.CompilerParams(needs_layout_passes=False),
        scratch_types=[
            pltpu.VMEM((_NCHUNK, _CHUNK), jnp.int32),
            pltpu.VMEM((_NCHUNK, _CHUNK), jnp.int32),
            pltpu.VMEM((2, 2 * _PPR, _D, 128), jnp.float32),
            pltpu.VMEM((_BPW // 8, 128), jnp.float32),
            pltpu.VMEM((_BPW,), jnp.float32),
            pltpu.VMEM((16,), jnp.float32),
            pltpu.SemaphoreType.DMA,
            pltpu.SemaphoreType.DMA,
        ],
    )
    tab_t = jnp.swapaxes(table, 0, 1)
    ni = node_i.reshape(_NW, _NCHUNK, _CHUNK)
    nj = node_j.reshape(_NW, _NCHUNK, _CHUNK)
    beta_vec = jnp.broadcast_to(beta.astype(jnp.float32), (16,))
    return k(tab_t, ni, nj, beta_vec)


# speedup vs baseline: 2.2640x; 1.0740x over previous
import jax
import jax.numpy as jnp
from jax import lax
from jax.experimental import pallas as pl
from jax.experimental.pallas import tpu as pltpu
from jax.experimental.pallas import tpu_sc as plsc

_NC = 2
_NS = 16
_NW = _NC * _NS
_B = 16384
_D = 32
_BPW = _B // _NW
_CHUNK = 128
_NCHUNK = _BPW // _CHUNK
_PPR = 4
_NROUND = _BPW // _PPR


def _sc_entry(tab_hbm, ni_hbm, nj_hbm, beta_hbm, out_hbm,
              idx_i, idx_j, blk, sums_v, out_v, beta_v, sem0, sem1, sem2):
    cid = lax.axis_index("c")
    sid = lax.axis_index("s")
    wid = sid * _NC + cid
    base = wid * _BPW

    pltpu.sync_copy(ni_hbm.at[wid], idx_i)
    pltpu.sync_copy(nj_hbm.at[wid], idx_j)
    pltpu.sync_copy(beta_hbm, beta_v)

    sems = (sem0, sem1, sem2)
    lane = jax.lax.broadcasted_iota(jnp.int32, (16,), 0)

    def loadidx(g):
        c = lax.div(g, _CHUNK // _PPR)
        lo = lax.rem(g, _CHUNK // _PPR) * _PPR
        iv = idx_i[c, pl.ds(lo, 16)]
        jv = idx_j[c, pl.ds(lo, 16)]
        return iv, jv

    def fire(g, k):
        iv, jv = loadidx(g)
        tc_i = lax.shift_right_logical(iv, 7) * 128
        tc_j = lax.shift_right_logical(jv, 7) * 128
        for p in range(_PPR):
            pltpu.async_copy(
                tab_hbm.at[:, pl.ds(pl.multiple_of(tc_i[p], 128), 128)],
                blk.at[k, 2 * p], sems[k])
            pltpu.async_copy(
                tab_hbm.at[:, pl.ds(pl.multiple_of(tc_j[p], 128), 128)],
                blk.at[k, 2 * p + 1], sems[k])

    def consume(g, k):
        for p in range(2 * _PPR):
            pltpu.make_async_copy(
                tab_hbm.at[:, pl.ds(0, 128)], blk.at[k, p], sems[k]).wait()
        iv, jv = loadidx(g)
        col_i = iv & 127
        col_j = jv & 127
        for p in range(_PPR):
            ci = jnp.broadcast_to(col_i[p], (16,))
            cj = jnp.broadcast_to(col_j[p], (16,))
            zi0 = plsc.load_gather(blk.at[k, 2 * p], [lane, ci])
            zi1 = plsc.load_gather(blk.at[k, 2 * p], [lane + 16, ci])
            zj0 = plsc.load_gather(blk.at[k, 2 * p + 1], [lane, cj])
            zj1 = plsc.load_gather(blk.at[k, 2 * p + 1], [lane + 16, cj])
            d0 = zi0 - zj0
            d1 = zi1 - zj1
            s2 = d0 * d0 + d1 * d1
            rr = g * _PPR + p
            sums_v[lax.shift_right_logical(rr, 3),
                   pl.ds((rr & 7) * 16, 16)] = plsc.cumsum(s2)

    fire(0, 0)
    fire(1, 1)

    def round_triple(q, carry):
        g = q * 3
        fire(g + 2, 2)
        consume(g, 0)

        @pl.when(g + 3 < _NROUND)
        def _():
            fire(g + 3, 0)

        consume(g + 1, 1)

        @pl.when(g + 4 < _NROUND)
        def _():
            fire(g + 4, 1)

        consume(g + 2, 2)
        return carry

    lax.fori_loop(0, _NROUND // 3, round_triple, 0)
    consume(_NROUND - 2, 0)
    consume(_NROUND - 1, 1)

    beta_vec = beta_v[...]
    last = jnp.full((16,), 15, jnp.int32)
    for g in range(_BPW // 16):
        rows = g * 16 + lane
        x = jnp.maximum(
            plsc.load_gather(
                sums_v,
                [lax.shift_right_logical(rows, 3), (rows & 7) * 16 + last]),
            1e-12)
        i = plsc.bitcast(x, jnp.int32)
        i = 0x5F3759DF - lax.shift_right_arithmetic(i, 1)
        r = plsc.bitcast(i, jnp.float32)
        half = 0.5 * x
        for _ in range(3):
            r = r * (1.5 - half * r * r)
        out_v[pl.ds(g * 16, 16)] = beta_vec - x * r

    pltpu.sync_copy(out_v, out_hbm.at[pl.ds(base, _BPW)])


def kernel(node_i, node_j, table, beta):
    mesh = plsc.VectorSubcoreMesh(core_axis_name="c", subcore_axis_name="s")
    k = pl.kernel(
        _sc_entry,
        out_type=jax.ShapeDtypeStruct((_B,), jnp.float32),
        mesh=mesh,
        compiler_params=pltpu.CompilerParams(needs_layout_passes=False),
        scratch_types=[
            pltpu.VMEM((_NCHUNK, _CHUNK), jnp.int32),
            pltpu.VMEM((_NCHUNK, _CHUNK), jnp.int32),
            pltpu.VMEM((3, 2 * _PPR, _D, 128), jnp.float32),
            pltpu.VMEM((_BPW // 8, 128), jnp.float32),
            pltpu.VMEM((_BPW,), jnp.float32),
            pltpu.VMEM((16,), jnp.float32),
            pltpu.SemaphoreType.DMA,
            pltpu.SemaphoreType.DMA,
            pltpu.SemaphoreType.DMA,
        ],
    )
    tab_t = jnp.swapaxes(table, 0, 1)
    ni = node_i.reshape(_NW, _NCHUNK, _CHUNK)
    nj = node_j.reshape(_NW, _NCHUNK, _CHUNK)
    beta_vec = jnp.broadcast_to(beta.astype(jnp.float32), (16,))
    return k(tab_t, ni, nj, beta_vec)


# speedup vs baseline: 2.2811x; 1.0076x over previous
import jax
import jax.numpy as jnp
from jax import lax
from jax.experimental import pallas as pl
from jax.experimental.pallas import tpu as pltpu
from jax.experimental.pallas import tpu_sc as plsc

_NC = 2
_NS = 16
_NW = _NC * _NS
_B = 16384
_D = 32
_BPW = _B // _NW
_CHUNK = 128
_NCHUNK = _BPW // _CHUNK
_PPR = 4
_NROUND = _BPW // _PPR


def _sc_entry(tab_hbm, ni_hbm, nj_hbm, beta_hbm, out_hbm,
              idx_i, idx_j, blk, sums_v, out_v, beta_v, sem0, sem1, sem2):
    cid = lax.axis_index("c")
    sid = lax.axis_index("s")
    wid = sid * _NC + cid
    base = wid * _BPW

    pltpu.sync_copy(ni_hbm.at[wid], idx_i)
    pltpu.sync_copy(nj_hbm.at[wid], idx_j)
    pltpu.sync_copy(beta_hbm, beta_v)

    sems = (sem0, sem1, sem2)
    lane = jax.lax.broadcasted_iota(jnp.int32, (16,), 0)

    def loadidx(g):
        c = lax.div(g, _CHUNK // _PPR)
        lo = lax.rem(g, _CHUNK // _PPR) * _PPR
        iv = idx_i[c, pl.ds(lo, 16)]
        jv = idx_j[c, pl.ds(lo, 16)]
        return iv, jv

    def fire(g, k):
        iv, jv = loadidx(g)
        tc_i = lax.shift_right_logical(iv, 7) * 128
        tc_j = lax.shift_right_logical(jv, 7) * 128
        for p in range(_PPR):
            for tr in range(4):
                pltpu.async_copy(
                    tab_hbm.at[tr, :,
                               pl.ds(pl.multiple_of(tc_i[p], 128), 128)],
                    blk.at[k, 2 * p, tr], sems[k])
                pltpu.async_copy(
                    tab_hbm.at[tr, :,
                               pl.ds(pl.multiple_of(tc_j[p], 128), 128)],
                    blk.at[k, 2 * p + 1, tr], sems[k])

    def consume(g, k):
        for p in range(2 * _PPR):
            for tr in range(4):
                pltpu.make_async_copy(
                    tab_hbm.at[0, :, pl.ds(0, 128)], blk.at[k, p, tr],
                    sems[k]).wait()
        iv, jv = loadidx(g)
        col_i = iv & 127
        col_j = jv & 127
        for p in range(_PPR):
            ci = jnp.broadcast_to(col_i[p], (16,))
            cj = jnp.broadcast_to(col_j[p], (16,))
            tr_lo = lax.shift_right_logical(lane, 3)
            sub = lane & 7
            zi0 = plsc.load_gather(blk.at[k, 2 * p], [tr_lo, sub, ci])
            zi1 = plsc.load_gather(blk.at[k, 2 * p], [tr_lo + 2, sub, ci])
            zj0 = plsc.load_gather(blk.at[k, 2 * p + 1], [tr_lo, sub, cj])
            zj1 = plsc.load_gather(blk.at[k, 2 * p + 1], [tr_lo + 2, sub, cj])
            d0 = zi0 - zj0
            d1 = zi1 - zj1
            s2 = d0 * d0 + d1 * d1
            rr = g * _PPR + p
            sums_v[lax.shift_right_logical(rr, 3),
                   pl.ds((rr & 7) * 16, 16)] = plsc.cumsum(s2)

    fire(0, 0)
    fire(1, 1)

    def round_triple(q, carry):
        g = q * 3
        fire(g + 2, 2)
        consume(g, 0)

        @pl.when(g + 3 < _NROUND)
        def _():
            fire(g + 3, 0)

        consume(g + 1, 1)

        @pl.when(g + 4 < _NROUND)
        def _():
            fire(g + 4, 1)

        consume(g + 2, 2)
        return carry

    lax.fori_loop(0, _NROUND // 3, round_triple, 0)
    consume(_NROUND - 2, 0)
    consume(_NROUND - 1, 1)

    beta_vec = beta_v[...]
    last = jnp.full((16,), 15, jnp.int32)
    for g in range(_BPW // 16):
        rows = g * 16 + lane
        x = jnp.maximum(
            plsc.load_gather(
                sums_v,
                [lax.shift_right_logical(rows, 3), (rows & 7) * 16 + last]),
            1e-12)
        i = plsc.bitcast(x, jnp.int32)
        i = 0x5F3759DF - lax.shift_right_arithmetic(i, 1)
        r = plsc.bitcast(i, jnp.float32)
        half = 0.5 * x
        for _ in range(3):
            r = r * (1.5 - half * r * r)
        out_v[pl.ds(g * 16, 16)] = beta_vec - x * r

    pltpu.sync_copy(out_v, out_hbm.at[pl.ds(base, _BPW)])


def kernel(node_i, node_j, table, beta):
    mesh = plsc.VectorSubcoreMesh(core_axis_name="c", subcore_axis_name="s")
    k = pl.kernel(
        _sc_entry,
        out_type=jax.ShapeDtypeStruct((_B,), jnp.float32),
        mesh=mesh,
        compiler_params=pltpu.CompilerParams(needs_layout_passes=False),
        scratch_types=[
            pltpu.VMEM((_NCHUNK, _CHUNK), jnp.int32),
            pltpu.VMEM((_NCHUNK, _CHUNK), jnp.int32),
            pltpu.VMEM((3, 2 * _PPR, 4, 8, 128), jnp.float32),
            pltpu.VMEM((_BPW // 8, 128), jnp.float32),
            pltpu.VMEM((_BPW,), jnp.float32),
            pltpu.VMEM((16,), jnp.float32),
            pltpu.SemaphoreType.DMA,
            pltpu.SemaphoreType.DMA,
            pltpu.SemaphoreType.DMA,
        ],
    )
    tab_t = jnp.swapaxes(table, 0, 1).reshape(4, 8, 1000000)
    ni = node_i.reshape(_NW, _NCHUNK, _CHUNK)
    nj = node_j.reshape(_NW, _NCHUNK, _CHUNK)
    beta_vec = jnp.broadcast_to(beta.astype(jnp.float32), (16,))
    return k(tab_t, ni, nj, beta_vec)
